# baseline (speedup 1.0000x reference)
import functools

import jax
import jax.numpy as jnp
from jax import lax
from jax.experimental import pallas as pl
from jax.experimental.pallas import tpu as pltpu
from jax.experimental.pallas import tpu_sc as plsc

N_NODES = 100000
N_PAIRS = 3200000
NC = 2
NS = 16
NW = NC * NS

PAIRS_PER_W = N_PAIRS // NW
C = 200
NCHUNK = PAIRS_PER_W // C
ROWS = 2 * C

NG = N_NODES // 2
STAGE = 3128

_mesh = plsc.VectorSubcoreMesh(core_axis_name="c", subcore_axis_name="s")


@functools.partial(
    pl.kernel,
    mesh=_mesh,
    out_type=jax.ShapeDtypeStruct((NW, 16), jnp.float32),
    scratch_types=[
        pltpu.VMEM_SHARED((NG, 32), jnp.float32),
        pltpu.VMEM((2, ROWS), jnp.int32),
        pltpu.VMEM((2, ROWS), jnp.int32),
        pltpu.VMEM((2, ROWS, 32), jnp.float32),
        pltpu.VMEM((16,), jnp.float32),
        pltpu.SemaphoreType.DMA,
        pltpu.SemaphoreType.DMA,
    ],
    compiler_params=pltpu.CompilerParams(use_tc_tiling_on_sc=False),
)
def _smoothness_kernel(x2_hbm, nbr_hbm, out_hbm, x2_spmem, idx_v, g_v,
                       rows_v, acc_v, sem0, sem1):
    cid = lax.axis_index("c")
    sid = lax.axis_index("s")
    wid = sid * NC + cid
    base_row = wid * (2 * PAIRS_PER_W)
    sems = (sem0, sem1)

    start = jnp.minimum(sid * STAGE, NG - STAGE)
    start = pl.multiple_of(start, 8)
    pltpu.sync_copy(x2_hbm.at[pl.ds(start, STAGE)],
                    x2_spmem.at[pl.ds(start, STAGE)])
    plsc.subcore_barrier()

    def fetch(c_i, b):
        off = pl.multiple_of(base_row + c_i * ROWS, 8)
        pltpu.sync_copy(nbr_hbm.at[pl.ds(off, ROWS)], idx_v.at[b])
        for m in range(ROWS // 16):
            g_v[b, pl.ds(16 * m, 16)] = (
                idx_v[b, pl.ds(16 * m, 16)] >> 1)
        pltpu.async_copy(x2_spmem.at[g_v.at[b]], rows_v.at[b], sems[b])

    def drain(b):
        pltpu.make_async_copy(x2_spmem.at[g_v.at[b]], rows_v.at[b],
                              sems[b]).wait()

    fetch(0, 0)

    def step(t, acc):
        for b in (0, 1):
            c_i = 2 * t + b

            @pl.when(c_i + 1 < NCHUNK)
            def _():
                fetch(c_i + 1, 1 - b)

            drain(b)

            def pair_body(t8, a):
                ids = idx_v[b, pl.ds(16 * t8, 16)]
                parf = (ids & 1).astype(jnp.float32)
                for u in range(8):
                    k = 8 * t8 + u
                    pi = parf.at[jnp.full((16,), 2 * u, jnp.int32)].get(
                        mode="promise_in_bounds")
                    pj = parf.at[jnp.full((16,), 2 * u + 1, jnp.int32)].get(
                        mode="promise_in_bounds")
                    lo0 = rows_v[b, 2 * k, pl.ds(0, 16)]
                    hi0 = rows_v[b, 2 * k, pl.ds(16, 16)]
                    lo1 = rows_v[b, 2 * k + 1, pl.ds(0, 16)]
                    hi1 = rows_v[b, 2 * k + 1, pl.ds(16, 16)]
                    r0 = lo0 + pi * (hi0 - lo0)
                    r1 = lo1 + pj * (hi1 - lo1)
                    d = r0 - r1
                    a = a + d * d
                return a

            acc = lax.fori_loop(0, C // 8, pair_body, acc)
        return acc

    acc = lax.fori_loop(0, NCHUNK // 2, step,
                        jnp.zeros((16,), jnp.float32))
    acc_v[...] = acc
    pltpu.sync_copy(acc_v, out_hbm.at[wid])


def kernel(A, all_neighbors):
    x2 = A.reshape(NG, 32)
    nbr = all_neighbors.reshape(-1)
    partial = _smoothness_kernel(x2, nbr)
    return jnp.sum(partial)


# speedup vs baseline: 13.8894x; 13.8894x over previous
import functools

import jax
import jax.numpy as jnp
from jax import lax
from jax.experimental import pallas as pl
from jax.experimental.pallas import tpu as pltpu
from jax.experimental.pallas import tpu_sc as plsc

N_NODES = 100000
N_PAIRS = 3200000
NC = 2
NS = 16
NW = NC * NS

PAIRS_PER_W = N_PAIRS // NW
C = 1000
NCHUNK = PAIRS_PER_W // C

_mesh = plsc.VectorSubcoreMesh(core_axis_name="c", subcore_axis_name="s")


@functools.partial(
    pl.kernel,
    mesh=_mesh,
    out_type=jax.ShapeDtypeStruct((NW, 16), jnp.float32),
    scratch_types=[
        pltpu.VMEM((2, C), jnp.int32),
        pltpu.VMEM((2, C), jnp.int32),
        pltpu.VMEM((2, C, 16), jnp.float32),
        pltpu.VMEM((2, C, 16), jnp.float32),
        pltpu.VMEM((16,), jnp.float32),
        pltpu.SemaphoreType.DMA,
        pltpu.SemaphoreType.DMA,
    ],
    compiler_params=pltpu.CompilerParams(use_tc_tiling_on_sc=False),
)
def _smoothness_kernel(x_hbm, icol_hbm, jcol_hbm, out_hbm, idxi_v, idxj_v,
                       rowsi_v, rowsj_v, acc_v, sem0, sem1):
    wid = lax.axis_index("s") * NC + lax.axis_index("c")
    base = wid * PAIRS_PER_W
    sems = (sem0, sem1)

    def fetch(c_i, b):
        off = pl.multiple_of(base + c_i * C, 8)
        pltpu.sync_copy(icol_hbm.at[pl.ds(off, C)], idxi_v.at[b])
        pltpu.sync_copy(jcol_hbm.at[pl.ds(off, C)], idxj_v.at[b])
        pltpu.async_copy(x_hbm.at[idxi_v.at[b]], rowsi_v.at[b], sems[b])
        pltpu.async_copy(x_hbm.at[idxj_v.at[b]], rowsj_v.at[b], sems[b])

    def drain(b):
        pltpu.make_async_copy(x_hbm.at[idxi_v.at[b]], rowsi_v.at[b],
                              sems[b]).wait()
        pltpu.make_async_copy(x_hbm.at[idxj_v.at[b]], rowsj_v.at[b],
                              sems[b]).wait()

    fetch(0, 0)

    def step(t, acc):
        for b in (0, 1):
            c_i = 2 * t + b

            @pl.when(c_i + 1 < NCHUNK)
            def _():
                fetch(c_i + 1, 1 - b)

            drain(b)

            def pair_body(k, a):
                d = rowsi_v[b, k] - rowsj_v[b, k]
                return a + d * d

            acc = lax.fori_loop(0, C, pair_body, acc, unroll=8)
        return acc

    acc = lax.fori_loop(0, NCHUNK // 2, step,
                        jnp.zeros((16,), jnp.float32))
    acc_v[...] = acc
    pltpu.sync_copy(acc_v, out_hbm.at[wid])


def kernel(A, all_neighbors):
    x = A.reshape(N_NODES, 16)
    icol = all_neighbors[:, 0]
    jcol = all_neighbors[:, 1]
    partial = _smoothness_kernel(x, icol, jcol)
    return jnp.sum(partial)


# speedup vs baseline: 13.8979x; 1.0006x over previous
import functools

import jax
import jax.numpy as jnp
from jax import lax
from jax.experimental import pallas as pl
from jax.experimental.pallas import tpu as pltpu
from jax.experimental.pallas import tpu_sc as plsc

N_NODES = 100000
N_PAIRS = 3200000
NC = 2
NS = 16
NW = NC * NS

PAIRS_PER_W = N_PAIRS // NW
C = 1000
NCHUNK = PAIRS_PER_W // C

_mesh = plsc.VectorSubcoreMesh(core_axis_name="c", subcore_axis_name="s")


@functools.partial(
    pl.kernel,
    mesh=_mesh,
    out_type=jax.ShapeDtypeStruct((NW, 16), jnp.float32),
    scratch_types=[
        pltpu.VMEM((2, C), jnp.int32),
        pltpu.VMEM((2, C), jnp.int32),
        pltpu.VMEM((2, C, 16), jnp.float32),
        pltpu.VMEM((2, C, 16), jnp.float32),
        pltpu.VMEM((16,), jnp.float32),
        pltpu.SemaphoreType.DMA,
        pltpu.SemaphoreType.DMA,
    ],
    compiler_params=pltpu.CompilerParams(use_tc_tiling_on_sc=False),
)
def _smoothness_kernel(x_hbm, icol_hbm, jcol_hbm, out_hbm, idxi_v, idxj_v,
                       rowsi_v, rowsj_v, acc_v, sem0, sem1):
    wid = lax.axis_index("s") * NC + lax.axis_index("c")
    base = wid * PAIRS_PER_W
    sems = (sem0, sem1)

    def fetch(c_i, b):
        off = pl.multiple_of(base + c_i * C, 8)
        pltpu.sync_copy(icol_hbm.at[pl.ds(off, C)], idxi_v.at[b])
        pltpu.sync_copy(jcol_hbm.at[pl.ds(off, C)], idxj_v.at[b])
        pltpu.async_copy(x_hbm.at[idxi_v.at[b]], rowsi_v.at[b], sems[b])
        pltpu.async_copy(x_hbm.at[idxj_v.at[b]], rowsj_v.at[b], sems[b])

    def drain(b):
        pltpu.make_async_copy(x_hbm.at[idxi_v.at[b]], rowsi_v.at[b],
                              sems[b]).wait()
        pltpu.make_async_copy(x_hbm.at[idxj_v.at[b]], rowsj_v.at[b],
                              sems[b]).wait()

    fetch(0, 0)

    def step(t, acc):
        for b in (0, 1):
            c_i = 2 * t + b

            @pl.when(c_i + 1 < NCHUNK)
            def _():
                fetch(c_i + 1, 1 - b)

            drain(b)

            def pair_body(k, a):
                d = rowsi_v[b, k] - rowsj_v[b, k]
                return a + d * d

            acc = lax.fori_loop(0, C, pair_body, acc, unroll=8)
        return acc

    acc = lax.fori_loop(0, NCHUNK // 2, step,
                        jnp.zeros((16,), jnp.float32))
    acc_v[...] = acc
    pltpu.sync_copy(acc_v, out_hbm.at[wid])


def kernel(A, all_neighbors):
    x = (A + jnp.float32(0.0)).reshape(N_NODES, 16)
    icol = all_neighbors[:, 0]
    jcol = all_neighbors[:, 1]
    partial = _smoothness_kernel(x, icol, jcol)
    return jnp.sum(partial)
